# baseline (speedup 1.0000x reference)
import functools

import jax
import jax.numpy as jnp
from jax import lax
from jax.experimental import pallas as pl
from jax.experimental.pallas import tpu as pltpu
from jax.experimental.pallas import tpu_sc as plsc

B, L = 1024, 200
NUCL_DIM, SPEC_DIM = 128, 64
OUT_DIM = NUCL_DIM + 2 * SPEC_DIM
NW = 32
BPW = B // NW
CHUNK_A, CHUNK_B = 128, L - 128
ROWS_PER_STEP = 4


def _body(seq_ref, pbsf_ref, rtf_ref, nucl_ref, pbst_ref, rtt_ref,
          out_ref,
          pbst_v, rtt_v, pbsf_v, rtf_v, seq_all, rb0, rb1,
          sg0, sg1, so0, so1):
    wid = lax.axis_index("s") * 2 + lax.axis_index("c")
    base = wid * BPW

    pltpu.sync_copy(pbst_ref, pbst_v)
    pltpu.sync_copy(rtt_ref, rtt_v)
    pltpu.sync_copy(pbsf_ref.at[pl.ds(base, BPW)], pbsf_v)
    pltpu.sync_copy(rtf_ref.at[pl.ds(base, BPW)], rtf_v)
    pltpu.sync_copy(seq_ref.at[pl.ds(base * L, BPW * L)], seq_all)

    half = jnp.full((16,), 0.5, jnp.float32)
    lane_ids = lax.iota(jnp.int32, 16)
    pvs = [pbsf_v[pl.ds(16 * k, 16)] for k in range(BPW // 16)]
    rvs = [rtf_v[pl.ds(16 * k, 16)] for k in range(BPW // 16)]

    def spec_rows(j):
        sel = lane_ids == (j % 16)
        zero = jnp.float32(0.0)
        pval = jnp.full((16,), jnp.sum(jnp.where(sel, pvs[j // 16], zero)))
        rval = jnp.full((16,), jnp.sum(jnp.where(sel, rvs[j // 16], zero)))
        regs = []
        for k in range(SPEC_DIM // 16):
            regs.append(jnp.where(pval > half, pbst_v[1, pl.ds(16 * k, 16)],
                                  pbst_v[0, pl.ds(16 * k, 16)]))
        for k in range(SPEC_DIM // 16):
            regs.append(jnp.where(rval > half, rtt_v[1, pl.ds(16 * k, 16)],
                                  rtt_v[0, pl.ds(16 * k, 16)]))
        return regs

    rbs, sgs, sos = (rb0, rb1), (sg0, sg1), (so0, so1)
    g_handles = [None] * BPW
    o_handles = [None] * BPW

    def fire_gathers(j):
        rb = rbs[j % 2]
        return (
            pltpu.async_copy(
                nucl_ref.at[seq_all.at[pl.ds(j * L, CHUNK_A)]],
                rb.at[pl.ds(0, CHUNK_A), pl.ds(0, NUCL_DIM)], sgs[j % 2]),
            pltpu.async_copy(
                nucl_ref.at[seq_all.at[pl.ds(j * L + CHUNK_A, CHUNK_B)]],
                rb.at[pl.ds(CHUNK_A, CHUNK_B), pl.ds(0, NUCL_DIM)], sgs[j % 2]),
        )

    def fill_spec(j):
        rb = rbs[j % 2]
        regs = spec_rows(j)

        def step(i, carry):
            r0 = i * ROWS_PER_STEP
            for dr in range(ROWS_PER_STEP):
                for k in range(len(regs)):
                    rb[r0 + dr, pl.ds(NUCL_DIM + 16 * k, 16)] = regs[k]
            return carry

        lax.fori_loop(0, L // ROWS_PER_STEP, step, 0)

    for j in range(BPW):
        if j >= 2:
            o_handles[j - 2].wait()
        g_handles[j] = fire_gathers(j)
        fill_spec(j)
        if j >= 1:
            for h in g_handles[j - 1]:
                h.wait()
            o_handles[j - 1] = pltpu.async_copy(
                rbs[(j - 1) % 2], out_ref.at[pl.ds((base + j - 1) * L, L)],
                sos[(j - 1) % 2])
    for h in g_handles[BPW - 1]:
        h.wait()
    o_handles[BPW - 1] = pltpu.async_copy(
        rbs[(BPW - 1) % 2], out_ref.at[pl.ds((base + BPW - 1) * L, L)],
        sos[(BPW - 1) % 2])
    o_handles[BPW - 2].wait()
    o_handles[BPW - 1].wait()


def kernel(seq, pbs_feat, rt_feat, nucl_table, pbs_table, rt_table):
    mesh = plsc.VectorSubcoreMesh(core_axis_name="c", subcore_axis_name="s")
    run = functools.partial(
        pl.kernel,
        mesh=mesh,
        compiler_params=pltpu.CompilerParams(needs_layout_passes=False),
        out_type=jax.ShapeDtypeStruct((B * L, OUT_DIM), jnp.float32),
        scratch_types=[
            pltpu.VMEM((2, SPEC_DIM), jnp.float32),
            pltpu.VMEM((2, SPEC_DIM), jnp.float32),
            pltpu.VMEM((BPW,), jnp.float32),
            pltpu.VMEM((BPW,), jnp.float32),
            pltpu.VMEM((BPW * L,), jnp.int32),
            pltpu.VMEM((L, OUT_DIM), jnp.float32),
            pltpu.VMEM((L, OUT_DIM), jnp.float32),
            pltpu.SemaphoreType.DMA,
            pltpu.SemaphoreType.DMA,
            pltpu.SemaphoreType.DMA,
            pltpu.SemaphoreType.DMA,
        ],
    )(_body)
    out = run(seq.reshape(B * L), pbs_feat, rt_feat,
              nucl_table, pbs_table, rt_table)
    return out.reshape(B, L, OUT_DIM)


# speedup vs baseline: 5.1922x; 5.1922x over previous
import functools

import jax
import jax.numpy as jnp
from jax import lax
from jax.experimental import pallas as pl
from jax.experimental.pallas import tpu as pltpu
from jax.experimental.pallas import tpu_sc as plsc

B, L = 1024, 200
NUCL_DIM, SPEC_DIM = 128, 64
OUT_DIM = NUCL_DIM + 2 * SPEC_DIM
NW = 32
BPW = B // NW
CHUNK_A, CHUNK_B = 128, L - 128
ROWS_PER_STEP = 4


def _body(seq_ref, pbsf_ref, rtf_ref, nucl_ref, pbst_ref, rtt_ref,
          out_ref, ntab_hbm,
          nucl_v, pbst_v, rtt_v, pbsf_v, rtf_v, seq_all, rb0, rb1,
          sg0, sg1, so0, so1):
    wid = lax.axis_index("s") * 2 + lax.axis_index("c")
    base = wid * BPW

    pltpu.sync_copy(nucl_ref, nucl_v.at[pl.ds(0, 6)])
    for k in range(NUCL_DIM // 16):
        nucl_v[6, pl.ds(16 * k, 16)] = jnp.zeros((16,), jnp.float32)
        nucl_v[7, pl.ds(16 * k, 16)] = jnp.zeros((16,), jnp.float32)
    pltpu.sync_copy(nucl_v, ntab_hbm.at[pl.ds(wid * 8, 8)])
    pltpu.sync_copy(pbst_ref, pbst_v)
    pltpu.sync_copy(rtt_ref, rtt_v)
    pltpu.sync_copy(pbsf_ref.at[pl.ds(base, BPW)], pbsf_v)
    pltpu.sync_copy(rtf_ref.at[pl.ds(base, BPW)], rtf_v)
    pltpu.sync_copy(seq_ref.at[pl.ds(base * L, BPW * L)], seq_all)

    half = jnp.full((16,), 0.5, jnp.float32)
    lane_ids = lax.iota(jnp.int32, 16)
    pvs = [pbsf_v[pl.ds(16 * k, 16)] for k in range(BPW // 16)]
    rvs = [rtf_v[pl.ds(16 * k, 16)] for k in range(BPW // 16)]

    def spec_rows(j):
        sel = lane_ids == (j % 16)
        zero = jnp.float32(0.0)
        pval = jnp.full((16,), jnp.sum(jnp.where(sel, pvs[j // 16], zero)))
        rval = jnp.full((16,), jnp.sum(jnp.where(sel, rvs[j // 16], zero)))
        regs = []
        for k in range(SPEC_DIM // 16):
            regs.append(jnp.where(pval > half, pbst_v[1, pl.ds(16 * k, 16)],
                                  pbst_v[0, pl.ds(16 * k, 16)]))
        for k in range(SPEC_DIM // 16):
            regs.append(jnp.where(rval > half, rtt_v[1, pl.ds(16 * k, 16)],
                                  rtt_v[0, pl.ds(16 * k, 16)]))
        return regs

    rbs, sgs, sos = (rb0, rb1), (sg0, sg1), (so0, so1)
    g_handles = [None] * BPW
    o_handles = [None] * BPW

    def fire_gathers(j):
        rb = rbs[j % 2]
        return (
            pltpu.async_copy(
                ntab_hbm.at[pl.ds(wid * 8, 8)].at[seq_all.at[pl.ds(j * L, CHUNK_A)]],
                rb.at[pl.ds(0, CHUNK_A), pl.ds(0, NUCL_DIM)], sgs[j % 2]),
            pltpu.async_copy(
                ntab_hbm.at[pl.ds(wid * 8, 8)].at[seq_all.at[pl.ds(j * L + CHUNK_A, CHUNK_B)]],
                rb.at[pl.ds(CHUNK_A, CHUNK_B), pl.ds(0, NUCL_DIM)], sgs[j % 2]),
        )

    def fill_spec(j):
        rb = rbs[j % 2]
        regs = spec_rows(j)

        def step(i, carry):
            r0 = i * ROWS_PER_STEP
            for dr in range(ROWS_PER_STEP):
                for k in range(len(regs)):
                    rb[r0 + dr, pl.ds(NUCL_DIM + 16 * k, 16)] = regs[k]
            return carry

        lax.fori_loop(0, L // ROWS_PER_STEP, step, 0)

    for j in range(BPW):
        if j >= 2:
            o_handles[j - 2].wait()
        g_handles[j] = fire_gathers(j)
        fill_spec(j)
        if j >= 1:
            for h in g_handles[j - 1]:
                h.wait()
            o_handles[j - 1] = pltpu.async_copy(
                rbs[(j - 1) % 2], out_ref.at[pl.ds((base + j - 1) * L, L)],
                sos[(j - 1) % 2])
    for h in g_handles[BPW - 1]:
        h.wait()
    o_handles[BPW - 1] = pltpu.async_copy(
        rbs[(BPW - 1) % 2], out_ref.at[pl.ds((base + BPW - 1) * L, L)],
        sos[(BPW - 1) % 2])
    o_handles[BPW - 2].wait()
    o_handles[BPW - 1].wait()


def kernel(seq, pbs_feat, rt_feat, nucl_table, pbs_table, rt_table):
    mesh = plsc.VectorSubcoreMesh(core_axis_name="c", subcore_axis_name="s")
    run = functools.partial(
        pl.kernel,
        mesh=mesh,
        compiler_params=pltpu.CompilerParams(needs_layout_passes=False),
        out_type=[
            jax.ShapeDtypeStruct((B * L, OUT_DIM), jnp.float32),
            jax.ShapeDtypeStruct((NW * 8, NUCL_DIM), jnp.float32),
        ],
        scratch_types=[
            pltpu.VMEM((8, NUCL_DIM), jnp.float32),
            pltpu.VMEM((2, SPEC_DIM), jnp.float32),
            pltpu.VMEM((2, SPEC_DIM), jnp.float32),
            pltpu.VMEM((BPW,), jnp.float32),
            pltpu.VMEM((BPW,), jnp.float32),
            pltpu.VMEM((BPW * L,), jnp.int32),
            pltpu.VMEM((L, OUT_DIM), jnp.float32),
            pltpu.VMEM((L, OUT_DIM), jnp.float32),
            pltpu.SemaphoreType.DMA,
            pltpu.SemaphoreType.DMA,
            pltpu.SemaphoreType.DMA,
            pltpu.SemaphoreType.DMA,
        ],
    )(_body)
    out, _ = run(seq.reshape(B * L), pbs_feat, rt_feat,
                 nucl_table, pbs_table, rt_table)
    return out.reshape(B, L, OUT_DIM)
